# baseline (speedup 1.0000x reference)
import functools

import numpy as np
import jax
import jax.numpy as jnp
from jax import lax
from jax.experimental import pallas as pl
from jax.experimental.pallas import tpu as pltpu
from jax.experimental.pallas import tpu_sc as plsc

POS_THRESH = 0.1
NEG_THRESH = 1.4
NUM_POS = 5192
NUM_HN_SAMPLES = 2048

NW = 32
PCH = 88
NPCH = 2
PB = PCH * NPCH
M_PAD = NW * PB
BLK_M = 704
K_PAD = 10240


@functools.lru_cache(maxsize=None)
def _selections(N0, N1, n_pairs):
    rng = np.random.RandomState(0)
    sel0 = rng.choice(N0, min(N0, NUM_HN_SAMPLES), replace=False)
    sel1 = rng.choice(N1, min(N1, NUM_HN_SAMPLES), replace=False)
    if n_pairs > NUM_POS:
        pos_sel = rng.choice(n_pairs, NUM_POS, replace=False)
    else:
        pos_sel = np.arange(n_pairs)
    return sel0.astype(np.int32), sel1.astype(np.int32), pos_sel.astype(np.int32)


def _sc_gather_fn(n_sub):
    sb = n_sub // NW
    mesh = plsc.VectorSubcoreMesh(core_axis_name="c", subcore_axis_name="s")
    out_type = [
        jax.ShapeDtypeStruct((M_PAD, 128), jnp.float32),
        jax.ShapeDtypeStruct((M_PAD, 128), jnp.float32),
        jax.ShapeDtypeStruct((n_sub, 128), jnp.float32),
        jax.ShapeDtypeStruct((n_sub, 128), jnp.float32),
        jax.ShapeDtypeStruct((NW, NPCH, PCH), jnp.int32),
        jax.ShapeDtypeStruct((NW, NPCH, PCH), jnp.int32),
    ]
    scratch = [
        pltpu.VMEM((NPCH, PCH), jnp.int32),
        pltpu.VMEM((NPCH, PCH), jnp.int32),
        pltpu.VMEM((PCH, 128), jnp.float32),
        pltpu.VMEM((sb,), jnp.int32),
        pltpu.VMEM((sb, 128), jnp.float32),
        pltpu.SemaphoreType.DMA,
    ]

    @functools.partial(pl.kernel, mesh=mesh, out_type=out_type,
                       scratch_types=scratch)
    def k(f0_h, f1_h, mflat_h, off0_h, off1_h, s0_h, s1_h,
          posf0_o, posf1_o, subf0_o, subf1_o, pi0_o, pi1_o,
          offv, pidxv, prows, sidxv, srows, sem):
        wid = lax.axis_index("s") * 2 + lax.axis_index("c")
        pbase = wid * PB
        sbase = wid * sb

        def pos_side(off_h, table_h, pi_o, posf_o):
            pltpu.sync_copy(off_h.at[wid], offv)
            for c in range(NPCH):
                pltpu.async_copy(mflat_h.at[offv.at[c]], pidxv.at[c], sem).wait()
            pltpu.sync_copy(pidxv, pi_o.at[wid])
            for c in range(NPCH):
                pltpu.async_copy(table_h.at[pidxv.at[c]], prows, sem).wait()
                pltpu.sync_copy(prows, posf_o.at[pl.ds(pbase + c * PCH, PCH)])

        pos_side(off0_h, f0_h, pi0_o, posf0_o)
        pos_side(off1_h, f1_h, pi1_o, posf1_o)

        def sub_side(s_h, table_h, subf_o):
            pltpu.sync_copy(s_h.at[wid], sidxv)
            pltpu.async_copy(table_h.at[sidxv], srows, sem).wait()
            pltpu.sync_copy(srows, subf_o.at[pl.ds(sbase, sb)])

        sub_side(s0_h, f0_h, subf0_o)
        sub_side(s1_h, f1_h, subf1_o)

    return k


def _tc_loss_kernel(posf0_ref, posf1_ref, subf0_ref, subf1_ref,
                    pi0_ref, pi1_ref, m0_ref, m1_ref,
                    sel0_ref, sel1_ref, out_ref, acc_ref,
                    *, hash_seed, n_valid, n_sub, grid_m):
    p = pl.program_id(0)

    @pl.when(p == 0)
    def _init():
        for i in range(5):
            acc_ref[i] = 0.0

    a0 = posf0_ref[...]
    a1 = posf1_ref[...]
    b0 = subf0_ref[...]
    b1 = subf1_ref[...]

    ones = jnp.ones((1, 128), jnp.float32)
    dot = functools.partial(
        lax.dot_general,
        dimension_numbers=(((1,), (1,)), ((), ())),
        preferred_element_type=jnp.float32,
        precision=lax.Precision.HIGHEST,
    )

    a0sq = jnp.sum(a0 * a0, axis=1, keepdims=True)
    a1sq = jnp.sum(a1 * a1, axis=1, keepdims=True)
    b0sq = dot(ones, b0 * b0)
    b1sq = dot(ones, b1 * b1)

    iota = lax.broadcasted_iota(jnp.int32, (BLK_M, n_sub), 1)
    rows = p * BLK_M + lax.broadcasted_iota(jnp.int32, (BLK_M, 1), 0)
    valid = rows < n_valid

    def side(aq, asq, bsq, bmat, sel_row, pidx):
        d2 = jnp.maximum(asq + bsq - 2.0 * dot(aq, bmat), 0.0)
        dmin = jnp.min(d2, axis=1, keepdims=True)
        is_min = d2 == dmin
        jpos = jnp.min(jnp.where(is_min, iota, n_sub), axis=1, keepdims=True)
        selval = jnp.sum(jnp.where(iota == jpos, sel_row, 0),
                         axis=1, keepdims=True)
        dist = jnp.sqrt(dmin + 1e-07)
        nl = jnp.square(jnp.maximum(NEG_THRESH - dist, 0.0))
        return nl, selval, pidx

    nl0, selval0, pi0 = side(a0, a0sq, b1sq, b1, sel1_ref[...], pi0_ref[...])
    nl1, selval1, pi1 = side(a1, a1sq, b0sq, b0, sel0_ref[...], pi1_ref[...])

    neg_keys0 = pi0 + selval0 * hash_seed
    neg_keys1 = selval1 + pi1 * hash_seed
    pos_keys = m0_ref[...] + m1_ref[...] * hash_seed

    hit0 = jnp.sum((neg_keys0 == pos_keys).astype(jnp.int32),
                   axis=1, keepdims=True) > 0
    hit1 = jnp.sum((neg_keys1 == pos_keys).astype(jnp.int32),
                   axis=1, keepdims=True) > 0
    mask0 = valid & jnp.logical_not(hit0)
    mask1 = valid & jnp.logical_not(hit1)

    dpos = a0 - a1
    pos_sq = jnp.sum(dpos * dpos, axis=1, keepdims=True)
    pos_term = jnp.where(valid, jnp.maximum(pos_sq - POS_THRESH, 0.0), 0.0)

    acc_ref[0] += jnp.sum(pos_term)
    acc_ref[1] += jnp.sum(jnp.where(mask0, nl0, 0.0))
    acc_ref[2] += jnp.sum(mask0.astype(jnp.float32))
    acc_ref[3] += jnp.sum(jnp.where(mask1, nl1, 0.0))
    acc_ref[4] += jnp.sum(mask1.astype(jnp.float32))

    @pl.when(p == grid_m - 1)
    def _fin():
        pos_loss = acc_ref[0] / n_valid
        neg0 = acc_ref[1] / jnp.maximum(acc_ref[2], 1.0)
        neg1 = acc_ref[3] / jnp.maximum(acc_ref[4], 1.0)
        out_ref[0, 0] = pos_loss + (neg0 + neg1) / 2.0


def _tc_loss(posF0, posF1, subF0, subF1, pi0, pi1, m0, m1, sel0r, sel1r,
             hash_seed, n_valid):
    n_sub = subF0.shape[0]
    grid_m = M_PAD // BLK_M
    kern = functools.partial(
        _tc_loss_kernel, hash_seed=hash_seed, n_valid=n_valid,
        n_sub=n_sub, grid_m=grid_m)
    full = lambda shape: pl.BlockSpec(shape, lambda p: (0, 0))
    out = pl.pallas_call(
        kern,
        grid=(grid_m,),
        in_specs=[
            pl.BlockSpec((BLK_M, 128), lambda p: (p, 0)),
            pl.BlockSpec((BLK_M, 128), lambda p: (p, 0)),
            full((n_sub, 128)),
            full((n_sub, 128)),
            pl.BlockSpec((BLK_M, 1), lambda p: (p, 0)),
            pl.BlockSpec((BLK_M, 1), lambda p: (p, 0)),
            full((1, K_PAD)),
            full((1, K_PAD)),
            full((1, n_sub)),
            full((1, n_sub)),
        ],
        out_specs=pl.BlockSpec(memory_space=pltpu.SMEM),
        out_shape=jax.ShapeDtypeStruct((1, 1), jnp.float32),
        scratch_shapes=[pltpu.SMEM((8,), jnp.float32)],
        compiler_params=pltpu.CompilerParams(
            dimension_semantics=("arbitrary",)),
    )(posF0, posF1, subF0, subF1, pi0, pi1, m0, m1, sel0r, sel1r)
    return out[0, 0]


def kernel(F0, F1, matches):
    N0, N1 = int(F0.shape[0]), int(F1.shape[0])
    n_pairs = int(matches.shape[0])
    hash_seed = max(N0, N1)
    sel0, sel1, pos_sel = _selections(N0, N1, n_pairs)
    n_valid = len(pos_sel)
    n_sub = len(sel0)

    pos_pad = np.zeros(M_PAD, np.int32)
    pos_pad[:n_valid] = pos_sel
    off0 = (2 * pos_pad).reshape(NW, NPCH, PCH)
    off1 = (2 * pos_pad + 1).reshape(NW, NPCH, PCH)
    s0w = sel0.reshape(NW, n_sub // NW)
    s1w = sel1.reshape(NW, n_sub // NW)

    matches = matches.astype(jnp.int32)
    mflat = matches.reshape(-1)

    posF0, posF1, subF0, subF1, pi0, pi1 = _sc_gather_fn(n_sub)(
        F0, F1, mflat,
        jnp.asarray(off0), jnp.asarray(off1),
        jnp.asarray(s0w), jnp.asarray(s1w))

    pi0 = pi0.reshape(M_PAD, 1)
    pi1 = pi1.reshape(M_PAD, 1)

    m0 = jnp.full((1, K_PAD), -1, jnp.int32).at[0, :n_pairs].set(matches[:, 0])
    m1 = jnp.zeros((1, K_PAD), jnp.int32).at[0, :n_pairs].set(matches[:, 1])
    sel0r = jnp.asarray(sel0).reshape(1, n_sub)
    sel1r = jnp.asarray(sel1).reshape(1, n_sub)

    return _tc_loss(posF0, posF1, subF0, subF1, pi0, pi1, m0, m1,
                    sel0r, sel1r, hash_seed, n_valid)


# speedup vs baseline: 1.4299x; 1.4299x over previous
import functools

import numpy as np
import jax
import jax.numpy as jnp
from jax import lax
from jax.experimental import pallas as pl
from jax.experimental.pallas import tpu as pltpu
from jax.experimental.pallas import tpu_sc as plsc

POS_THRESH = 0.1
NEG_THRESH = 1.4
NUM_POS = 5192
NUM_HN_SAMPLES = 2048

NW = 32
PCH = 88
NPCH = 2
PB = PCH * NPCH
M_PAD = NW * PB
BLK_M = 704
K_PAD = 10240


@functools.lru_cache(maxsize=None)
def _selections(N0, N1, n_pairs):
    rng = np.random.RandomState(0)
    sel0 = rng.choice(N0, min(N0, NUM_HN_SAMPLES), replace=False)
    sel1 = rng.choice(N1, min(N1, NUM_HN_SAMPLES), replace=False)
    if n_pairs > NUM_POS:
        pos_sel = rng.choice(n_pairs, NUM_POS, replace=False)
    else:
        pos_sel = np.arange(n_pairs)
    return sel0.astype(np.int32), sel1.astype(np.int32), pos_sel.astype(np.int32)


def _sc_gather_fn(n_sub):
    sb = n_sub // NW
    mesh = plsc.VectorSubcoreMesh(core_axis_name="c", subcore_axis_name="s")
    out_type = [
        jax.ShapeDtypeStruct((M_PAD, 128), jnp.float32),
        jax.ShapeDtypeStruct((M_PAD, 128), jnp.float32),
        jax.ShapeDtypeStruct((n_sub, 128), jnp.float32),
        jax.ShapeDtypeStruct((n_sub, 128), jnp.float32),
        jax.ShapeDtypeStruct((NW, NPCH, PCH), jnp.int32),
        jax.ShapeDtypeStruct((NW, NPCH, PCH), jnp.int32),
    ]
    scratch = [
        pltpu.VMEM((NPCH, PCH), jnp.int32),
        pltpu.VMEM((NPCH, PCH), jnp.int32),
        pltpu.VMEM((NPCH, PCH), jnp.int32),
        pltpu.VMEM((NPCH, PCH), jnp.int32),
        pltpu.VMEM((2 * NPCH, PCH, 128), jnp.float32),
        pltpu.VMEM((sb,), jnp.int32),
        pltpu.VMEM((sb,), jnp.int32),
        pltpu.VMEM((sb, 128), jnp.float32),
        pltpu.VMEM((sb, 128), jnp.float32),
        pltpu.SemaphoreType.DMA,
        pltpu.SemaphoreType.DMA,
        pltpu.SemaphoreType.DMA,
        pltpu.SemaphoreType.DMA,
        pltpu.SemaphoreType.DMA,
    ]

    @functools.partial(pl.kernel, mesh=mesh, out_type=out_type,
                       scratch_types=scratch)
    def k(f0_h, f1_h, mflat_h, off0_h, off1_h, s0_h, s1_h,
          posf0_o, posf1_o, subf0_o, subf1_o, pi0_o, pi1_o,
          off0v, off1v, pidx0v, pidx1v, prows, sidx0v, sidx1v,
          srows0, srows1, sem_i, sem_m, sem_s, sem_f, sem_o):
        wid = lax.axis_index("s") * 2 + lax.axis_index("c")
        pbase = wid * PB
        sbase = wid * sb

        h_idx = [
            pltpu.async_copy(off0_h.at[wid], off0v, sem_i),
            pltpu.async_copy(off1_h.at[wid], off1v, sem_i),
            pltpu.async_copy(s0_h.at[wid], sidx0v, sem_i),
            pltpu.async_copy(s1_h.at[wid], sidx1v, sem_i),
        ]
        for h in h_idx:
            h.wait()

        h_m = []
        for c in range(NPCH):
            h_m.append(pltpu.async_copy(mflat_h.at[off0v.at[c]],
                                        pidx0v.at[c], sem_m))
            h_m.append(pltpu.async_copy(mflat_h.at[off1v.at[c]],
                                        pidx1v.at[c], sem_m))
        h_s0 = pltpu.async_copy(f0_h.at[sidx0v], srows0, sem_s)
        h_s1 = pltpu.async_copy(f1_h.at[sidx1v], srows1, sem_s)
        for h in h_m:
            h.wait()

        h_f = []
        for c in range(NPCH):
            h_f.append(pltpu.async_copy(f0_h.at[pidx0v.at[c]],
                                        prows.at[c], sem_f))
            h_f.append(pltpu.async_copy(f1_h.at[pidx1v.at[c]],
                                        prows.at[NPCH + c], sem_f))
        h_o = [
            pltpu.async_copy(pidx0v, pi0_o.at[wid], sem_o),
            pltpu.async_copy(pidx1v, pi1_o.at[wid], sem_o),
        ]
        h_s0.wait()
        h_s1.wait()
        h_o.append(pltpu.async_copy(srows0, subf0_o.at[pl.ds(sbase, sb)], sem_o))
        h_o.append(pltpu.async_copy(srows1, subf1_o.at[pl.ds(sbase, sb)], sem_o))
        for c, h in enumerate(h_f):
            h.wait()
        for c in range(NPCH):
            h_o.append(pltpu.async_copy(
                prows.at[c], posf0_o.at[pl.ds(pbase + c * PCH, PCH)], sem_o))
            h_o.append(pltpu.async_copy(
                prows.at[NPCH + c], posf1_o.at[pl.ds(pbase + c * PCH, PCH)], sem_o))
        for h in h_o:
            h.wait()

    return k


def _tc_loss_kernel(posf0_ref, posf1_ref, subf0_ref, subf1_ref,
                    pi0_ref, pi1_ref, m0_ref, m1_ref,
                    sel0_ref, sel1_ref, out_ref, acc_ref,
                    *, hash_seed, n_valid, n_sub, grid_m):
    p = pl.program_id(0)

    @pl.when(p == 0)
    def _init():
        for i in range(5):
            acc_ref[i] = 0.0

    a0 = posf0_ref[...]
    a1 = posf1_ref[...]
    b0 = subf0_ref[...]
    b1 = subf1_ref[...]

    ones = jnp.ones((1, 128), jnp.float32)
    dotf = functools.partial(
        lax.dot_general,
        dimension_numbers=(((1,), (1,)), ((), ())),
        preferred_element_type=jnp.float32,
        precision=lax.Precision.HIGHEST,
    )
    dotb = functools.partial(
        lax.dot_general,
        dimension_numbers=(((1,), (1,)), ((), ())),
        preferred_element_type=jnp.float32,
    )

    a0sq = jnp.sum(a0 * a0, axis=1, keepdims=True)
    a1sq = jnp.sum(a1 * a1, axis=1, keepdims=True)
    b0sq = dotf(ones, b0 * b0)
    b1sq = dotf(ones, b1 * b1)

    rows = p * BLK_M + lax.broadcasted_iota(jnp.int32, (BLK_M, 1), 0)
    valid = rows < n_valid

    def side(aq, asq, bsq, bmat, kc_row):
        g = dotb(aq.astype(jnp.bfloat16), bmat.astype(jnp.bfloat16))
        d2 = jnp.maximum(asq + bsq - 2.0 * g, 0.0)
        dmin = jnp.min(d2, axis=1, keepdims=True)
        kmin = jnp.min(jnp.where(d2 == dmin, kc_row, jnp.int32(0x7FFFFFFF)),
                       axis=1, keepdims=True)
        selval = jnp.bitwise_and(kmin, 32767)
        dist = jnp.sqrt(dmin + 1e-07)
        nl = jnp.square(jnp.maximum(NEG_THRESH - dist, 0.0))
        return nl, selval

    nl0, selval0 = side(a0, a0sq, b1sq, b1, sel1_ref[...])
    nl1, selval1 = side(a1, a1sq, b0sq, b0, sel0_ref[...])
    pi0 = pi0_ref[...]
    pi1 = pi1_ref[...]

    neg_keys0 = pi0 + selval0 * hash_seed
    neg_keys1 = selval1 + pi1 * hash_seed
    pos_keys = m0_ref[...] + m1_ref[...] * hash_seed

    hit0 = jnp.sum((neg_keys0 == pos_keys).astype(jnp.int32),
                   axis=1, keepdims=True) > 0
    hit1 = jnp.sum((neg_keys1 == pos_keys).astype(jnp.int32),
                   axis=1, keepdims=True) > 0
    mask0 = valid & jnp.logical_not(hit0)
    mask1 = valid & jnp.logical_not(hit1)

    dpos = a0 - a1
    pos_sq = jnp.sum(dpos * dpos, axis=1, keepdims=True)
    pos_term = jnp.where(valid, jnp.maximum(pos_sq - POS_THRESH, 0.0), 0.0)

    acc_ref[0] += jnp.sum(pos_term)
    acc_ref[1] += jnp.sum(jnp.where(mask0, nl0, 0.0))
    acc_ref[2] += jnp.sum(mask0.astype(jnp.float32))
    acc_ref[3] += jnp.sum(jnp.where(mask1, nl1, 0.0))
    acc_ref[4] += jnp.sum(mask1.astype(jnp.float32))

    @pl.when(p == grid_m - 1)
    def _fin():
        pos_loss = acc_ref[0] / n_valid
        neg0 = acc_ref[1] / jnp.maximum(acc_ref[2], 1.0)
        neg1 = acc_ref[3] / jnp.maximum(acc_ref[4], 1.0)
        out_ref[0, 0] = pos_loss + (neg0 + neg1) / 2.0


def _tc_loss(posF0, posF1, subF0, subF1, pi0, pi1, m0, m1, sel0r, sel1r,
             hash_seed, n_valid):
    n_sub = subF0.shape[0]
    grid_m = M_PAD // BLK_M
    kern = functools.partial(
        _tc_loss_kernel, hash_seed=hash_seed, n_valid=n_valid,
        n_sub=n_sub, grid_m=grid_m)
    full = lambda shape: pl.BlockSpec(shape, lambda p: (0, 0))
    out = pl.pallas_call(
        kern,
        grid=(grid_m,),
        in_specs=[
            pl.BlockSpec((BLK_M, 128), lambda p: (p, 0)),
            pl.BlockSpec((BLK_M, 128), lambda p: (p, 0)),
            full((n_sub, 128)),
            full((n_sub, 128)),
            pl.BlockSpec((BLK_M, 1), lambda p: (p, 0)),
            pl.BlockSpec((BLK_M, 1), lambda p: (p, 0)),
            full((1, K_PAD)),
            full((1, K_PAD)),
            full((1, n_sub)),
            full((1, n_sub)),
        ],
        out_specs=pl.BlockSpec(memory_space=pltpu.SMEM),
        out_shape=jax.ShapeDtypeStruct((1, 1), jnp.float32),
        scratch_shapes=[pltpu.SMEM((8,), jnp.float32)],
        compiler_params=pltpu.CompilerParams(
            dimension_semantics=("arbitrary",)),
    )(posF0, posF1, subF0, subF1, pi0, pi1, m0, m1, sel0r, sel1r)
    return out[0, 0]


def kernel(F0, F1, matches):
    N0, N1 = int(F0.shape[0]), int(F1.shape[0])
    n_pairs = int(matches.shape[0])
    hash_seed = max(N0, N1)
    sel0, sel1, pos_sel = _selections(N0, N1, n_pairs)
    n_valid = len(pos_sel)
    n_sub = len(sel0)

    pos_pad = np.zeros(M_PAD, np.int32)
    pos_pad[:n_valid] = pos_sel
    off0 = (2 * pos_pad).reshape(NW, NPCH, PCH)
    off1 = (2 * pos_pad + 1).reshape(NW, NPCH, PCH)
    s0w = sel0.reshape(NW, n_sub // NW)
    s1w = sel1.reshape(NW, n_sub // NW)

    matches = matches.astype(jnp.int32)
    mflat = matches.reshape(-1)

    posF0, posF1, subF0, subF1, pi0, pi1 = _sc_gather_fn(n_sub)(
        F0, F1, mflat,
        jnp.asarray(off0), jnp.asarray(off1),
        jnp.asarray(s0w), jnp.asarray(s1w))

    pi0 = pi0.reshape(M_PAD, 1)
    pi1 = pi1.reshape(M_PAD, 1)

    m0 = jnp.full((1, K_PAD), -1, jnp.int32).at[0, :n_pairs].set(matches[:, 0])
    m1 = jnp.zeros((1, K_PAD), jnp.int32).at[0, :n_pairs].set(matches[:, 1])
    jj = np.arange(n_sub, dtype=np.int32) * 32768
    sel0r = jnp.asarray((jj + sel0).reshape(1, n_sub))
    sel1r = jnp.asarray((jj + sel1).reshape(1, n_sub))

    return _tc_loss(posF0, posF1, subF0, subF1, pi0, pi1, m0, m1,
                    sel0r, sel1r, hash_seed, n_valid)
